# baseline (speedup 1.0000x reference)
import functools

import jax
import jax.numpy as jnp
from jax import lax
from jax.experimental import pallas as pl
from jax.experimental.pallas import tpu as pltpu
from jax.experimental.pallas import tpu_sc as plsc

NC = 2
NS = 16
LANES = 16
NBUF = 4


def _linear_body(x_ref, w_ref, b_ref, o_ref):
    h = lax.dot_general(
        x_ref[...], w_ref[...], (((1,), (1,)), ((), ())),
        preferred_element_type=jnp.float32) + b_ref[...]
    d2 = h.shape[1] // 2
    o_ref[0] = h[:, :d2]
    o_ref[1] = h[:, d2:]


def _make_scatter(n, d, chunks, k):
    dh = d // NC
    rps = n // NS
    mesh = plsc.VectorSubcoreMesh(
        core_axis_name="c", subcore_axis_name="s",
        num_cores=NC, num_subcores=NS)

    @functools.partial(
        pl.kernel,
        out_type=jax.ShapeDtypeStruct((n, d), jnp.float32),
        mesh=mesh,
        scratch_types=[
            pltpu.VMEM_SHARED((n, dh), jnp.float32),
            pltpu.VMEM((chunks, k), jnp.int32),
            pltpu.VMEM((chunks, k), jnp.int32),
            pltpu.VMEM((chunks, k), jnp.float32),
            pltpu.VMEM((NBUF, k, dh), jnp.float32),
            pltpu.SemaphoreType.DMA,
            pltpu.SemaphoreType.DMA,
            pltpu.SemaphoreType.DMA,
        ],
        compiler_params=pltpu.CompilerParams(
            use_tc_tiling_on_sc=False, needs_layout_passes=False),
    )
    def scatter(h2, colr, rowr, valr, zeros, out, acc, colv, rowv, valv,
                msg, sem_g, sem_s, sem_in):
        cid = lax.axis_index("c")
        sid = lax.axis_index("s")
        hc = h2.at[cid]

        pltpu.async_copy(colr.at[sid], colv, sem_in)
        pltpu.async_copy(rowr.at[sid], rowv, sem_in)
        pltpu.async_copy(valr.at[sid], valv, sem_in)
        pltpu.sync_copy(zeros.at[pl.ds(sid * rps, rps)],
                        acc.at[pl.ds(sid * rps, rps)])
        pltpu.make_async_copy(colr.at[sid], colv, sem_in).wait()
        pltpu.make_async_copy(rowr.at[sid], rowv, sem_in).wait()
        pltpu.make_async_copy(valr.at[sid], valv, sem_in).wait()
        plsc.subcore_barrier()

        def gather_wait():
            pltpu.make_async_copy(hc.at[colv.at[0]], msg.at[0], sem_g).wait()

        def scatter_wait():
            pltpu.make_async_copy(msg.at[0], acc.at[rowv.at[0]],
                                  sem_s).wait()

        pltpu.async_copy(hc.at[colv.at[0]], msg.at[0], sem_g)
        pltpu.async_copy(hc.at[colv.at[1]], msg.at[1], sem_g)

        def chunk_body(j, carry):
            b = lax.rem(j, NBUF)
            bn = lax.rem(j + 2, NBUF)
            @pl.when(j >= 2)
            def _():
                scatter_wait()

            @pl.when(j + 2 < chunks)
            def _():
                pltpu.async_copy(hc.at[colv.at[j + 2]], msg.at[bn], sem_g)

            gather_wait()
            mb = msg.at[b]
            vj = jnp.full((LANES,), j, jnp.int32)
            for i in range(k):
                vv = plsc.load_gather(
                    valv, [vj, jnp.full((LANES,), i, jnp.int32)])
                for f in range(dh // LANES):
                    sl = pl.ds(f * LANES, LANES)
                    mb[i, sl] = mb[i, sl] * vv
            pltpu.async_copy(mb, acc.at[rowv.at[j]], sem_s, add=True)
            return carry

        lax.fori_loop(0, chunks, chunk_body, 0)
        scatter_wait()
        scatter_wait()

        plsc.subcore_barrier()
        pltpu.sync_copy(acc.at[pl.ds(sid * rps, rps)],
                        out.at[pl.ds(sid * rps, rps),
                               pl.ds(cid * dh, dh)])

    return scatter


@jax.jit
def kernel(X, adj_indices, adj_values, W, b):
    n, d_in = X.shape
    d_out = W.shape[0]
    e = adj_values.shape[0]
    k = 80
    eps = -(-e // (NS * k)) * k
    chunks = eps // k
    pad = eps * NS - e

    row_blocks = 10
    rb = n // row_blocks
    h2 = pl.pallas_call(
        _linear_body,
        grid=(row_blocks,),
        in_specs=[
            pl.BlockSpec((rb, d_in), lambda i: (i, 0)),
            pl.BlockSpec((d_out, d_in), lambda i: (0, 0)),
            pl.BlockSpec((1, d_out), lambda i: (0, 0)),
        ],
        out_specs=pl.BlockSpec((NC, rb, d_out // NC), lambda i: (0, i, 0)),
        out_shape=jax.ShapeDtypeStruct((NC, n, d_out // NC), jnp.float32),
    )(X, W, b.reshape(1, d_out))

    colr = jnp.pad(adj_indices[1], (0, pad)).reshape(NS, chunks, k)
    rowr = jnp.pad(adj_indices[0], (0, pad)).reshape(NS, chunks, k)
    valr = jnp.pad(adj_values, (0, pad)).reshape(NS, chunks, k)
    zeros = jnp.zeros((n, d_out // NC), jnp.float32)

    return _make_scatter(n, d_out, chunks, k)(h2, colr, rowr, valr, zeros)


# speedup vs baseline: 1.2040x; 1.2040x over previous
import functools

import jax
import jax.numpy as jnp
from jax import lax
from jax.experimental import pallas as pl
from jax.experimental.pallas import tpu as pltpu
from jax.experimental.pallas import tpu_sc as plsc

NC = 2
NS = 16
LANES = 16
NBUF = 4


def _linear_body(x_ref, w_ref, b_ref, o_ref):
    h = lax.dot_general(
        x_ref[...], w_ref[...], (((1,), (1,)), ((), ())),
        preferred_element_type=jnp.float32) + b_ref[...]
    d2 = h.shape[1] // 2
    o_ref[0] = h[:, :d2]
    o_ref[1] = h[:, d2:]


def _make_scatter(n, d, chunks, k):
    dh = d // NC
    rps = n // NS
    mesh = plsc.VectorSubcoreMesh(
        core_axis_name="c", subcore_axis_name="s",
        num_cores=NC, num_subcores=NS)

    @functools.partial(
        pl.kernel,
        out_type=jax.ShapeDtypeStruct((n, d), jnp.float32),
        mesh=mesh,
        scratch_types=[
            pltpu.VMEM_SHARED((n, dh), jnp.float32),
            pltpu.VMEM((chunks, k), jnp.int32),
            pltpu.VMEM((chunks, k), jnp.int32),
            pltpu.VMEM((chunks, k), jnp.float32),
            pltpu.VMEM((NBUF, k, dh), jnp.float32),
            pltpu.SemaphoreType.DMA,
            pltpu.SemaphoreType.DMA,
            pltpu.SemaphoreType.DMA,
        ],
        compiler_params=pltpu.CompilerParams(
            use_tc_tiling_on_sc=False, needs_layout_passes=False),
    )
    def scatter(h2, colr, rowr, valr, zeros, out, acc, colv, rowv, valv,
                msg, sem_g, sem_s, sem_in):
        cid = lax.axis_index("c")
        sid = lax.axis_index("s")
        hc = h2.at[cid]

        pltpu.async_copy(colr.at[sid], colv, sem_in)
        pltpu.async_copy(rowr.at[sid], rowv, sem_in)
        pltpu.async_copy(valr.at[sid], valv, sem_in)
        pltpu.sync_copy(zeros.at[pl.ds(sid * rps, rps)],
                        acc.at[pl.ds(sid * rps, rps)])
        pltpu.make_async_copy(colr.at[sid], colv, sem_in).wait()
        pltpu.make_async_copy(rowr.at[sid], rowv, sem_in).wait()
        pltpu.make_async_copy(valr.at[sid], valv, sem_in).wait()
        plsc.subcore_barrier()

        def gather_wait():
            pltpu.make_async_copy(hc.at[colv.at[0]], msg.at[0], sem_g).wait()

        def scatter_wait():
            pltpu.make_async_copy(msg.at[0], acc.at[rowv.at[0]],
                                  sem_s).wait()

        pltpu.async_copy(hc.at[colv.at[0]], msg.at[0], sem_g)
        pltpu.async_copy(hc.at[colv.at[1]], msg.at[1], sem_g)

        def chunk_body(j, carry):
            b = lax.rem(j, NBUF)
            bn = lax.rem(j + 2, NBUF)
            @pl.when(j >= 2)
            def _():
                scatter_wait()

            @pl.when(j + 2 < chunks)
            def _():
                pltpu.async_copy(hc.at[colv.at[j + 2]], msg.at[bn], sem_g)

            gather_wait()
            mb = msg.at[b]
            for i16 in range(k // LANES):
                vals16 = valv[j, pl.ds(i16 * LANES, LANES)]
                for l in range(LANES):
                    i = i16 * LANES + l
                    vv = vals16.at[jnp.full((LANES,), l, jnp.int32)].get(
                        mode="promise_in_bounds")
                    for f in range(dh // LANES):
                        sl = pl.ds(f * LANES, LANES)
                        mb[i, sl] = mb[i, sl] * vv
            pltpu.async_copy(mb, acc.at[rowv.at[j]], sem_s, add=True)
            return carry

        lax.fori_loop(0, chunks, chunk_body, 0)
        scatter_wait()
        scatter_wait()

        plsc.subcore_barrier()
        pltpu.sync_copy(acc.at[pl.ds(sid * rps, rps)],
                        out.at[pl.ds(sid * rps, rps),
                               pl.ds(cid * dh, dh)])

    return scatter


@jax.jit
def kernel(X, adj_indices, adj_values, W, b):
    n, d_in = X.shape
    d_out = W.shape[0]
    e = adj_values.shape[0]
    k = 80
    eps = -(-e // (NS * k)) * k
    chunks = eps // k
    pad = eps * NS - e

    row_blocks = 10
    rb = n // row_blocks
    h2 = pl.pallas_call(
        _linear_body,
        grid=(row_blocks,),
        in_specs=[
            pl.BlockSpec((rb, d_in), lambda i: (i, 0)),
            pl.BlockSpec((d_out, d_in), lambda i: (0, 0)),
            pl.BlockSpec((1, d_out), lambda i: (0, 0)),
        ],
        out_specs=pl.BlockSpec((NC, rb, d_out // NC), lambda i: (0, i, 0)),
        out_shape=jax.ShapeDtypeStruct((NC, n, d_out // NC), jnp.float32),
    )(X, W, b.reshape(1, d_out))

    colr = jnp.pad(adj_indices[1], (0, pad)).reshape(NS, chunks, k)
    rowr = jnp.pad(adj_indices[0], (0, pad)).reshape(NS, chunks, k)
    valr = jnp.pad(adj_values, (0, pad)).reshape(NS, chunks, k)
    zeros = jnp.zeros((n, d_out // NC), jnp.float32)

    return _make_scatter(n, d_out, chunks, k)(h2, colr, rowr, valr, zeros)
